# Initial kernel scaffold; baseline (speedup 1.0000x reference)
import jax
import jax.numpy as jnp
from jax.experimental import pallas as pl


def kernel(pos, dir, intensity, t_matrix, W, decay, map_to_element, map_to_surface):
    raise NotImplementedError("write your pallas kernel here")


# speedup vs baseline: 3.1626x; 3.1626x over previous
import functools

import jax
import jax.numpy as jnp
from jax import lax
from jax.experimental import pallas as pl
from jax.experimental.pallas import tpu as pltpu
from jax.experimental.pallas import tpu_sc as plsc

N_RAYS = 32768
N_SURF = 64
NC = 2
NS = 16
NW = NC * NS
L = 16
R = N_RAYS // NW
G = R // L


def _scene_body(t_ref, pos_ref, dir_ref, int_ref, w_ref, dec_ref,
                opos_ref, odir_ref, oint_ref,
                t_v, pos_v, dir_v, int_v, w_v, dec_v,
                opos_v, odir_v, oint_v):
    wid = lax.axis_index("s") * NC + lax.axis_index("c")
    base = wid * R
    pltpu.sync_copy(t_ref.at[pl.ds(base * N_SURF, R * N_SURF)], t_v)
    pltpu.sync_copy(pos_ref.at[pl.ds(base * 3, R * 3)], pos_v)
    pltpu.sync_copy(dir_ref.at[pl.ds(base * 3, R * 3)], dir_v)
    pltpu.sync_copy(int_ref.at[pl.ds(base, R)], int_v)
    pltpu.sync_copy(w_ref, w_v)
    pltpu.sync_copy(dec_ref, dec_v)

    lane = lax.iota(jnp.int32, L)
    lane64 = lane * N_SURF
    lane3 = lane * 3
    inf = jnp.float32(jnp.inf)

    def group(g, carry):
        rows = g * L + lane
        rows64 = g * (L * N_SURF) + lane64
        rows3 = g * (L * 3) + lane3
        bt = jnp.full((L,), inf, dtype=jnp.float32)
        bi = jnp.zeros((L,), dtype=jnp.int32)
        for s in range(N_SURF):
            tv = plsc.load_gather(t_v, [rows64 + s])
            c = tv < bt
            bt = jnp.where(c, tv, bt)
            bi = jnp.where(c, jnp.int32(s), bi)
        bi9 = bi * 9
        wg = [plsc.load_gather(w_v, [bi9 + j]) for j in range(9)]
        dg = plsc.load_gather(dec_v, [bi])
        px = [plsc.load_gather(pos_v, [rows3 + c]) for c in range(3)]
        dx = [plsc.load_gather(dir_v, [rows3 + c]) for c in range(3)]
        it = plsc.load_gather(int_v, [rows])
        hit = (bt < inf) & (it > jnp.float32(0.0))
        op = [jnp.where(hit, px[c] + bt * dx[c], px[c]) for c in range(3)]
        od = [jnp.where(hit, dx[0] * wg[j] + dx[1] * wg[3 + j] + dx[2] * wg[6 + j],
                        dx[j]) for j in range(3)]
        oi = jnp.where(hit, it * dg, it)
        for c in range(3):
            plsc.store_scatter(opos_v, [rows3 + c], op[c])
            plsc.store_scatter(odir_v, [rows3 + c], od[c])
        plsc.store_scatter(oint_v, [rows], oi)
        return carry

    lax.fori_loop(0, G, group, 0)

    pltpu.sync_copy(opos_v, opos_ref.at[pl.ds(base * 3, R * 3)])
    pltpu.sync_copy(odir_v, odir_ref.at[pl.ds(base * 3, R * 3)])
    pltpu.sync_copy(oint_v, oint_ref.at[pl.ds(base, R)])


_scene_kernel = functools.partial(
    pl.kernel,
    out_type=(jax.ShapeDtypeStruct((N_RAYS * 3,), jnp.float32),
              jax.ShapeDtypeStruct((N_RAYS * 3,), jnp.float32),
              jax.ShapeDtypeStruct((N_RAYS,), jnp.float32)),
    scratch_types=[
        pltpu.VMEM((R * N_SURF,), jnp.float32),
        pltpu.VMEM((R * 3,), jnp.float32),
        pltpu.VMEM((R * 3,), jnp.float32),
        pltpu.VMEM((R,), jnp.float32),
        pltpu.VMEM((N_SURF * 9,), jnp.float32),
        pltpu.VMEM((N_SURF,), jnp.float32),
        pltpu.VMEM((R * 3,), jnp.float32),
        pltpu.VMEM((R * 3,), jnp.float32),
        pltpu.VMEM((R,), jnp.float32),
    ],
    mesh=plsc.VectorSubcoreMesh(core_axis_name="c", subcore_axis_name="s"),
    compiler_params=pltpu.CompilerParams(needs_layout_passes=False),
)(_scene_body)


def kernel(pos, dir, intensity, t_matrix, W, decay, map_to_element, map_to_surface):
    del map_to_element, map_to_surface
    opos, odir, oint = _scene_kernel(
        t_matrix.reshape(-1), pos.reshape(-1), dir.reshape(-1), intensity,
        W.reshape(-1), decay)
    return (opos.reshape(N_RAYS, 3), odir.reshape(N_RAYS, 3), oint)


# speedup vs baseline: 3.6713x; 1.1609x over previous
import functools

import jax
import jax.numpy as jnp
from jax import lax
from jax.experimental import pallas as pl
from jax.experimental.pallas import tpu as pltpu
from jax.experimental.pallas import tpu_sc as plsc

N_RAYS = 32768
N_SURF = 64
NC = 2
NS = 16
NW = NC * NS
L = 16
R = N_RAYS // NW
G = R // L


T_STRIDE = 65


def _scene_body(t_ref, pos_ref, dir_ref, int_ref, w_ref, dec_ref,
                opos_ref, odir_ref, oint_ref,
                t_v, pos_v, dir_v, int_v, w_v, dec_v,
                opos_v, odir_v, oint_v):
    wid = lax.axis_index("s") * NC + lax.axis_index("c")
    base = wid * R
    pltpu.sync_copy(t_ref.at[pl.ds(base, R)], t_v.at[:, pl.ds(0, N_SURF)])
    pltpu.sync_copy(pos_ref.at[pl.ds(base * 3, R * 3)], pos_v)
    pltpu.sync_copy(dir_ref.at[pl.ds(base * 3, R * 3)], dir_v)
    pltpu.sync_copy(int_ref.at[pl.ds(base, R)], int_v)
    pltpu.sync_copy(w_ref, w_v)
    pltpu.sync_copy(dec_ref, dec_v)

    lane = lax.iota(jnp.int32, L)
    lane3 = lane * 3
    inf = jnp.float32(jnp.inf)

    def group(g, carry):
        rows = g * L + lane
        rows3 = g * (L * 3) + lane3
        bts = [jnp.full((L,), inf, dtype=jnp.float32) for _ in range(4)]
        bis = [jnp.zeros((L,), dtype=jnp.int32) for _ in range(4)]
        for s in range(N_SURF):
            k = s % 4
            tv = plsc.load_gather(t_v, [rows, jnp.full((L,), s, jnp.int32)])
            c = tv < bts[k]
            bts[k] = jnp.where(c, tv, bts[k])
            bis[k] = jnp.where(c, jnp.int32(s), bis[k])

        def combine(ta, ia, tb, ib):
            c = (ta < tb) | ((ta == tb) & (ia < ib))
            return jnp.where(c, ta, tb), jnp.where(c, ia, ib)

        t01, i01 = combine(bts[0], bis[0], bts[1], bis[1])
        t23, i23 = combine(bts[2], bis[2], bts[3], bis[3])
        bt, bi = combine(t01, i01, t23, i23)
        bi9 = bi * 9
        wg = [plsc.load_gather(w_v, [bi9 + j]) for j in range(9)]
        dg = plsc.load_gather(dec_v, [bi])
        px = [plsc.load_gather(pos_v, [rows3 + c]) for c in range(3)]
        dx = [plsc.load_gather(dir_v, [rows3 + c]) for c in range(3)]
        it = plsc.load_gather(int_v, [rows])
        hit = (bt < inf) & (it > jnp.float32(0.0))
        op = [jnp.where(hit, px[c] + bt * dx[c], px[c]) for c in range(3)]
        od = [jnp.where(hit, dx[0] * wg[j] + dx[1] * wg[3 + j] + dx[2] * wg[6 + j],
                        dx[j]) for j in range(3)]
        oi = jnp.where(hit, it * dg, it)
        for c in range(3):
            plsc.store_scatter(opos_v, [rows3 + c], op[c])
            plsc.store_scatter(odir_v, [rows3 + c], od[c])
        plsc.store_scatter(oint_v, [rows], oi)
        return carry

    lax.fori_loop(0, G, group, 0)

    pltpu.sync_copy(opos_v, opos_ref.at[pl.ds(base * 3, R * 3)])
    pltpu.sync_copy(odir_v, odir_ref.at[pl.ds(base * 3, R * 3)])
    pltpu.sync_copy(oint_v, oint_ref.at[pl.ds(base, R)])


_scene_kernel = functools.partial(
    pl.kernel,
    out_type=(jax.ShapeDtypeStruct((N_RAYS * 3,), jnp.float32),
              jax.ShapeDtypeStruct((N_RAYS * 3,), jnp.float32),
              jax.ShapeDtypeStruct((N_RAYS,), jnp.float32)),
    scratch_types=[
        pltpu.VMEM((R, T_STRIDE), jnp.float32),
        pltpu.VMEM((R * 3,), jnp.float32),
        pltpu.VMEM((R * 3,), jnp.float32),
        pltpu.VMEM((R,), jnp.float32),
        pltpu.VMEM((N_SURF * 9,), jnp.float32),
        pltpu.VMEM((N_SURF,), jnp.float32),
        pltpu.VMEM((R * 3,), jnp.float32),
        pltpu.VMEM((R * 3,), jnp.float32),
        pltpu.VMEM((R,), jnp.float32),
    ],
    mesh=plsc.VectorSubcoreMesh(core_axis_name="c", subcore_axis_name="s"),
    compiler_params=pltpu.CompilerParams(needs_layout_passes=False,
                                         use_tc_tiling_on_sc=False),
)(_scene_body)


def kernel(pos, dir, intensity, t_matrix, W, decay, map_to_element, map_to_surface):
    del map_to_element, map_to_surface
    opos, odir, oint = _scene_kernel(
        t_matrix, pos.reshape(-1), dir.reshape(-1), intensity,
        W.reshape(-1), decay)
    return (opos.reshape(N_RAYS, 3), odir.reshape(N_RAYS, 3), oint)
